# Initial kernel scaffold; baseline (speedup 1.0000x reference)
import jax
import jax.numpy as jnp
from jax.experimental import pallas as pl


def kernel(predicted_edge_uvp, edge_index, face):
    raise NotImplementedError("write your pallas kernel here")


# speedup vs baseline: 2.6295x; 2.6295x over previous
import functools

import jax
import jax.numpy as jnp
from jax import lax
from jax.experimental import pallas as pl
from jax.experimental.pallas import tpu as pltpu
from jax.experimental.pallas import tpu_sc as plsc

_E = 1_600_000
_N = 100_000
_C = 200_000

_CHUNK = 1024
_NB = _CHUNK // 128
_NFULL = _E // _CHUNK
_EFULL = _NFULL * _CHUNK
_REM = _E - _EFULL
_REM_TILE = _NFULL % 32

_CB = 128
_CPAD = 64
_NCB = (_C + _CPAD) // _CB

_RPS = 6256


@functools.cache
def _build_sc_main():
  mesh = plsc.VectorSubcoreMesh(core_axis_name="c", subcore_axis_name="s")

  @functools.partial(
      pl.kernel,
      out_type=(
          jax.ShapeDtypeStruct((2, 4 * _N), jnp.float32),
          jax.ShapeDtypeStruct((3 * (_C + _CPAD),), jnp.float32),
      ),
      mesh=mesh,
      compiler_params=pltpu.CompilerParams(needs_layout_passes=False,
                                           use_tc_tiling_on_sc=False),
      scratch_types=[
          pltpu.VMEM_SHARED((_N, 8), jnp.float32),
          pltpu.VMEM((3 * _CHUNK,), jnp.float32),
          pltpu.VMEM((_CHUNK, 8), jnp.float32),
          pltpu.VMEM((_NB, 128), jnp.int32),
          pltpu.VMEM((_NB, 128), jnp.int32),
          pltpu.VMEM((128,), jnp.int32),
          pltpu.VMEM((128,), jnp.int32),
          pltpu.VMEM((128,), jnp.int32),
          pltpu.VMEM((3, 128), jnp.int32),
          pltpu.VMEM((3, 128), jnp.float32),
          pltpu.VMEM((3, 128), jnp.float32),
          pltpu.VMEM((3, 128), jnp.float32),
          pltpu.VMEM((3 * _CB,), jnp.float32),
          pltpu.VMEM((4096,), jnp.float32),
          pltpu.SemaphoreType.DMA,
      ],
  )
  def _sc_main(uvp, snd, rcv, snd_t, rcv_t, face0, face1, face2,
               partials, cell_out,
               acc, stage, payload, sidx, ridx, fi0, fi1, fi2, eidx,
               g0, g1, g2, cres, flat, sem):
    cid = lax.axis_index("c")
    sid = lax.axis_index("s")
    wid = cid * 16 + sid

    j16 = lax.iota(jnp.int32, 16)
    zeros = jnp.zeros((16,), jnp.float32)
    ones = jnp.ones((16,), jnp.float32)

    def _zero_payload(q, _):
        g = 16 * q + j16
        plsc.store_scatter(payload, [g // 8, g % 8], zeros)
        return 0
    lax.fori_loop(0, _CHUNK * 8 // 16, _zero_payload, 0)
    base = sid * _RPS
    for t in range(6):
        pltpu.sync_copy(payload, acc.at[pl.ds(base + t * _CHUNK, _CHUNK)])

    @pl.when(sid != 15)
    def _():
        pltpu.sync_copy(payload.at[pl.ds(0, 112)],
                        acc.at[pl.ds(base + 6 * _CHUNK, 112)])

    @pl.when(sid == 15)
    def _():
        pltpu.sync_copy(payload.at[pl.ds(0, 16)],
                        acc.at[pl.ds(base + 6 * _CHUNK, 16)])

    plsc.subcore_barrier()

    col3 = jnp.full((16,), 3, jnp.int32)

    def _set_ones(q, _):
        plsc.store_scatter(payload, [16 * q + j16, col3], ones)
        return 0
    lax.fori_loop(0, _CHUNK // 16, _set_ones, 0)

    rowoff = [(16 * k + j16) // 3 for k in range(3)]
    coloff = [(16 * k + j16) % 3 for k in range(3)]

    def _chunk(ci, n, tail):
        nb = n // 128
        pltpu.sync_copy(uvp.at[pl.ds(3 * _CHUNK * ci, 3 * n)],
                        stage.at[pl.ds(0, 3 * n)])
        if tail:
            pltpu.sync_copy(snd_t, sidx.at[pl.ds(0, nb)])
            pltpu.sync_copy(rcv_t, ridx.at[pl.ds(0, nb)])
        else:
            pltpu.sync_copy(snd.at[pl.ds(ci * _NB, nb)], sidx.at[pl.ds(0, nb)])
            pltpu.sync_copy(rcv.at[pl.ds(ci * _NB, nb)], ridx.at[pl.ds(0, nb)])

        def _repack(m, _):
            for k in range(3):
                v = stage[pl.ds(48 * m + 16 * k, 16)]
                plsc.store_scatter(payload, [16 * m + rowoff[k], coloff[k]], v)
            return 0
        lax.fori_loop(0, n // 16, _repack, 0)
        for b in range(nb):
            src = payload.at[pl.ds(b * 128, 128)]
            pltpu.sync_copy(src, acc.at[sidx.at[b]], add=True)
            pltpu.sync_copy(src, acc.at[ridx.at[b]], add=True)

    nfull = (jnp.int32(_NFULL + 31) - wid) // 32

    def _edge_body(k, _):
        _chunk(wid + 32 * k, _CHUNK, False)
        return 0
    lax.fori_loop(0, nfull, _edge_body, 0)

    @pl.when(wid == _REM_TILE)
    def _():
        _chunk(jnp.int32(_NFULL), _REM, True)

    def _cells(bi):
        c0 = bi * _CB
        fis = (fi0, fi1, fi2)
        faces = (face0, face1, face2)
        gs = (g0, g1, g2)
        for jf in range(3):
            pltpu.sync_copy(faces[jf].at[pl.ds(c0, _CB)], fis[jf])
        for jf in range(3):
            for c in range(3):
                def _bidx(q, _, jf=jf, c=c):
                    eidx[c, pl.ds(16 * q, 16)] = (
                        3 * fis[jf][pl.ds(16 * q, 16)] + c)
                    return 0
                lax.fori_loop(0, 8, _bidx, 0)
            for c in range(3):
                pltpu.async_copy(uvp.at[eidx.at[c]], gs[jf].at[c], sem).wait()

        def _avg(q, _):
            for c in range(3):
                sl = pl.ds(16 * q, 16)
                v = (g0[c, sl] + g1[c, sl] + g2[c, sl]) / 3.0
                plsc.store_scatter(cres, [48 * q + 3 * j16 + c], v)
            return 0
        lax.fori_loop(0, 8, _avg, 0)
        pltpu.sync_copy(cres, cell_out.at[pl.ds(3 * c0, 3 * _CB)])

    ncb = (jnp.int32(_NCB + 31) - wid) // 32

    def _cell_body(k, _):
        _cells(wid + 32 * k)
        return 0
    lax.fori_loop(0, ncb, _cell_body, 0)

    plsc.subcore_barrier()

    def _dump_block(r0, nrows):
        pltpu.sync_copy(acc.at[pl.ds(r0, nrows)], payload.at[pl.ds(0, nrows)])

        def _flatten(q, _):
            g = 16 * q + j16
            flat[pl.ds(16 * q, 16)] = plsc.load_gather(
                payload, [g // 4, g % 4])
            return 0
        lax.fori_loop(0, nrows * 4 // 16, _flatten, 0)
        pltpu.sync_copy(flat.at[pl.ds(0, 4 * nrows)],
                        partials.at[cid, pl.ds(4 * r0, 4 * nrows)])

    for t in range(6):
        _dump_block(base + t * _CHUNK, _CHUNK)

    @pl.when(sid != 15)
    def _():
        _dump_block(base + 6 * _CHUNK, 112)

    @pl.when(sid == 15)
    def _():
        _dump_block(base + 6 * _CHUNK, 16)

  return _sc_main


def _combine(p_ref, o_ref):
    s = p_ref[0] + p_ref[1]
    o_ref[...] = s[:, :3] / jnp.maximum(s[:, 3:4], 1.0)


def kernel(predicted_edge_uvp, edge_index, face):
    uvp_flat = predicted_edge_uvp.reshape(3 * _E)
    snd = edge_index[0, :_EFULL].reshape(_EFULL // 128, 128)
    rcv = edge_index[1, :_EFULL].reshape(_EFULL // 128, 128)
    snd_t = edge_index[0, _EFULL:].reshape(_REM // 128, 128)
    rcv_t = edge_index[1, _EFULL:].reshape(_REM // 128, 128)
    face_p = jnp.pad(face, ((0, 0), (0, _CPAD)))
    partials, cell_flat = _build_sc_main()(
        uvp_flat, snd, rcv, snd_t, rcv_t, face_p[0], face_p[1], face_p[2])
    cell = cell_flat[:3 * _C].reshape(_C, 3)
    p4 = partials.reshape(2, _N, 4)
    node = pl.pallas_call(
        _combine,
        grid=(50,),
        in_specs=[pl.BlockSpec((2, _N // 50, 4), lambda i: (0, i, 0))],
        out_specs=pl.BlockSpec((_N // 50, 3), lambda i: (i, 0)),
        out_shape=jax.ShapeDtypeStruct((_N, 3), jnp.float32),
    )(p4)
    return node, cell


# speedup vs baseline: 6.5874x; 2.5051x over previous
import functools

import jax
import jax.numpy as jnp
from jax import lax
from jax.experimental import pallas as pl
from jax.experimental.pallas import tpu as pltpu
from jax.experimental.pallas import tpu_sc as plsc

_E = 1_600_000
_N = 100_000
_C = 200_000

_CHUNK = 1024
_NB = _CHUNK // 128
_EPAD = 512
_EP = _E + _EPAD
_NCH = _EP // _CHUNK

_CB = 128
_CPAD = 64
_NCB = (_C + _CPAD) // _CB

_RPS = 6256


@functools.cache
def _build_sc_main():
  mesh = plsc.VectorSubcoreMesh(core_axis_name="c", subcore_axis_name="s")

  @functools.partial(
      pl.kernel,
      out_type=(
          jax.ShapeDtypeStruct((2, 4 * _N), jnp.float32),
          jax.ShapeDtypeStruct((3 * (_C + _CPAD),), jnp.float32),
      ),
      mesh=mesh,
      compiler_params=pltpu.CompilerParams(needs_layout_passes=False,
                                           use_tc_tiling_on_sc=False),
      scratch_types=[
          pltpu.VMEM_SHARED((_N, 8), jnp.float32),
          pltpu.VMEM((_NB, 128, 8), jnp.float32),
          pltpu.VMEM((_CHUNK, 8), jnp.float32),
          pltpu.VMEM((_NB, 128), jnp.int32),
          pltpu.VMEM((_NB, 128), jnp.int32),
          pltpu.VMEM((128,), jnp.int32),
          pltpu.VMEM((128,), jnp.int32),
          pltpu.VMEM((128,), jnp.int32),
          pltpu.VMEM((_CB, 8), jnp.float32),
          pltpu.VMEM((_CB, 8), jnp.float32),
          pltpu.VMEM((_CB, 8), jnp.float32),
          pltpu.VMEM((3 * _CB,), jnp.float32),
          pltpu.VMEM((4096,), jnp.float32),
          pltpu.SemaphoreType.DMA,
      ],
  )
  def _sc_main(pay3, pay2, ei3, face0, face1, face2,
               partials, cell_out,
               acc, stagep, payload, sidx, ridx, fi0, fi1, fi2,
               g0, g1, g2, cres, flat, sem):
    cid = lax.axis_index("c")
    sid = lax.axis_index("s")
    wid = cid * 16 + sid

    j16 = lax.iota(jnp.int32, 16)
    zeros = jnp.zeros((16,), jnp.float32)

    def _zero_payload(q, _):
        g = 16 * q + j16
        plsc.store_scatter(payload, [g // 8, g % 8], zeros)
        return 0
    lax.fori_loop(0, _CHUNK * 8 // 16, _zero_payload, 0)
    base = sid * _RPS
    for t in range(6):
        pltpu.sync_copy(payload, acc.at[pl.ds(base + t * _CHUNK, _CHUNK)])

    @pl.when(sid != 15)
    def _():
        pltpu.sync_copy(payload.at[pl.ds(0, 112)],
                        acc.at[pl.ds(base + 6 * _CHUNK, 112)])

    @pl.when(sid == 15)
    def _():
        pltpu.sync_copy(payload.at[pl.ds(0, 16)],
                        acc.at[pl.ds(base + 6 * _CHUNK, 16)])

    plsc.subcore_barrier()

    def _chunk(ci):
        r0 = ci * _NB
        pltpu.sync_copy(pay3.at[pl.ds(r0, _NB)], stagep)
        pltpu.sync_copy(ei3.at[0, pl.ds(r0, _NB)], sidx)
        pltpu.sync_copy(ei3.at[1, pl.ds(r0, _NB)], ridx)
        for b in range(_NB):
            src = stagep.at[b]
            pltpu.sync_copy(src, acc.at[sidx.at[b]], add=True)
            pltpu.sync_copy(src, acc.at[ridx.at[b]], add=True)

    nch = (jnp.int32(_NCH + 31) - wid) // 32

    def _edge_body(k, _):
        _chunk(wid + 32 * k)
        return 0
    lax.fori_loop(0, nch, _edge_body, 0)

    def _cells(bi):
        c0 = bi * _CB
        fis = (fi0, fi1, fi2)
        faces = (face0, face1, face2)
        gs = (g0, g1, g2)
        for jf in range(3):
            pltpu.sync_copy(faces[jf].at[pl.ds(c0, _CB)], fis[jf])
        for jf in range(3):
            pltpu.async_copy(pay2.at[fis[jf]], gs[jf], sem).wait()

        def _avg(q, _):
            rows = 16 * q + j16
            for c in range(3):
                cc = jnp.full((16,), c, jnp.int32)
                v = (plsc.load_gather(g0, [rows, cc])
                     + plsc.load_gather(g1, [rows, cc])
                     + plsc.load_gather(g2, [rows, cc])) / 3.0
                plsc.store_scatter(cres, [48 * q + 3 * j16 + c], v)
            return 0
        lax.fori_loop(0, 8, _avg, 0)
        pltpu.sync_copy(cres, cell_out.at[pl.ds(3 * c0, 3 * _CB)])

    ncb = (jnp.int32(_NCB + 31) - wid) // 32

    def _cell_body(k, _):
        _cells(wid + 32 * k)
        return 0
    lax.fori_loop(0, ncb, _cell_body, 0)

    plsc.subcore_barrier()

    def _dump_block(r0, nrows):
        pltpu.sync_copy(acc.at[pl.ds(r0, nrows)], payload.at[pl.ds(0, nrows)])

        def _flatten(q, _):
            g = 16 * q + j16
            flat[pl.ds(16 * q, 16)] = plsc.load_gather(
                payload, [g // 4, g % 4])
            return 0
        lax.fori_loop(0, nrows * 4 // 16, _flatten, 0)
        pltpu.sync_copy(flat.at[pl.ds(0, 4 * nrows)],
                        partials.at[cid, pl.ds(4 * r0, 4 * nrows)])

    for t in range(6):
        _dump_block(base + t * _CHUNK, _CHUNK)

    @pl.when(sid != 15)
    def _():
        _dump_block(base + 6 * _CHUNK, 112)

    @pl.when(sid == 15)
    def _():
        _dump_block(base + 6 * _CHUNK, 16)

  return _sc_main


def _combine(p_ref, o_ref):
    s = p_ref[0] + p_ref[1]
    o_ref[...] = s[:, :3] / jnp.maximum(s[:, 3:4], 1.0)


def kernel(predicted_edge_uvp, edge_index, face):
    pay = jnp.concatenate(
        [predicted_edge_uvp,
         jnp.ones((_E, 1), jnp.float32),
         jnp.zeros((_E, 4), jnp.float32)], axis=1)
    pay = jnp.pad(pay, ((0, _EPAD), (0, 0)))
    pay3 = pay.reshape(_EP // 128, 128, 8)
    pay2 = pay.reshape(_EP, 8)
    ei3 = jnp.pad(edge_index, ((0, 0), (0, _EPAD))).reshape(2, _EP // 128, 128)
    face_p = jnp.pad(face, ((0, 0), (0, _CPAD)))
    partials, cell_flat = _build_sc_main()(
        pay3, pay2, ei3, face_p[0], face_p[1], face_p[2])
    cell = cell_flat[:3 * _C].reshape(_C, 3)
    p4 = partials.reshape(2, _N, 4)
    node = pl.pallas_call(
        _combine,
        grid=(50,),
        in_specs=[pl.BlockSpec((2, _N // 50, 4), lambda i: (0, i, 0))],
        out_specs=pl.BlockSpec((_N // 50, 3), lambda i: (i, 0)),
        out_shape=jax.ShapeDtypeStruct((_N, 3), jnp.float32),
    )(p4)
    return node, cell
